# Initial kernel scaffold; baseline (speedup 1.0000x reference)
import jax
import jax.numpy as jnp
from jax.experimental import pallas as pl


def kernel(protein_embeddings, genomic_embeddings, params, amp2target_edges, target2amp_edges, sup_edges, neg_edges):
    raise NotImplementedError("write your pallas kernel here")


# speedup vs baseline: 60.9279x; 60.9279x over previous
import jax
import jax.numpy as jnp
import numpy as np
from jax.experimental import pallas as pl

H = 8
DH = 16
D = H * DH


def _lin_body(x_ref, w_ref, b_ref, o_ref):
    o_ref[...] = jnp.dot(x_ref[...], w_ref[...],
                         preferred_element_type=jnp.float32) + b_ref[...]


def _linear(x, W, b):
    M, K = x.shape
    N = W.shape[1]
    return pl.pallas_call(
        _lin_body,
        out_shape=jax.ShapeDtypeStruct((M, N), jnp.float32),
    )(x, W, b.reshape(1, N))


def _dist_body(xb_ref, emb_ref, sq_ref, o_ref):
    xb = xb_ref[...]
    emb = emb_ref[...]
    g = jax.lax.dot_general(xb, emb, (((1,), (1,)), ((), ())),
                            preferred_element_type=jnp.float32)
    sqb = jnp.sum(xb * xb, axis=1, keepdims=True)
    d2 = sqb + sq_ref[...] - 2.0 * g
    d = jnp.sqrt(jnp.maximum(d2, 0.0))
    tb = xb.shape[0]
    n = emb.shape[0]
    rows = jax.lax.broadcasted_iota(jnp.int32, (tb, n), 0) + pl.program_id(0) * tb
    cols = jax.lax.broadcasted_iota(jnp.int32, (tb, n), 1)
    o_ref[...] = jnp.where(rows == cols, 0.0, d)


def _pdist(emb):
    n, k = emb.shape
    tb = 256
    sq = jnp.sum(emb * emb, axis=1)[None, :]
    return pl.pallas_call(
        _dist_body,
        grid=(n // tb,),
        in_specs=[
            pl.BlockSpec((tb, k), lambda j: (j, 0)),
            pl.BlockSpec((n, k), lambda j: (0, 0)),
            pl.BlockSpec((1, n), lambda j: (0, 0)),
        ],
        out_specs=pl.BlockSpec((tb, n), lambda j: (j, 0)),
        out_shape=jax.ShapeDtypeStruct((n, n), jnp.float32),
    )(emb, emb, sq)


def _gat_body(fsT_ref, fs_ref, fd_ref, lwt_ref, at_ref, sel_ref, b_ref, o_ref):
    fsT = fsT_ref[...]
    fs = fs_ref[...]
    at = at_ref[...]
    sel = sel_ref[...]
    bias = b_ref[...]
    tv = fd_ref.shape[0]

    def step(vi, carry):
        fd_row = fd_ref[pl.ds(vi, 1), :]
        fd_col = fd_row.T
        x = fsT + fd_col
        y = jnp.maximum(x, 0.2 * x)
        logits = jnp.dot(at, y, preferred_element_type=jnp.float32)
        lw = lwt_ref[pl.ds(vi, 1), :]
        logits = logits + lw
        m = jnp.max(logits, axis=1, keepdims=True)
        m = jnp.where(m == -jnp.inf, 0.0, m)
        p = jnp.exp(logits - m)
        den = jnp.sum(p, axis=1, keepdims=True)
        msg = jnp.dot(p, fs, preferred_element_type=jnp.float32)
        num = jnp.sum(msg * sel, axis=0, keepdims=True)
        dsel = jnp.sum(den * sel, axis=0, keepdims=True)
        out_row = jnp.maximum(num / jnp.maximum(dsel, 1e-16) + bias, 0.0)
        o_ref[pl.ds(vi, 1), :] = out_row
        return carry

    jax.lax.fori_loop(0, tv, step, 0)


def _gat_dense(fs, fd, lwt, attn, bias):
    ns = fs.shape[0]
    nd = fd.shape[0]
    tv = 128
    at = attn.reshape(-1)[:, None] * (
        jnp.arange(D)[:, None] // DH == jnp.arange(H)[None, :]).astype(jnp.float32)
    at = at.T
    sel = (jnp.arange(D)[None, :] // DH == jnp.arange(H)[:, None]).astype(jnp.float32)
    return pl.pallas_call(
        _gat_body,
        grid=(nd // tv,),
        in_specs=[
            pl.BlockSpec((D, ns), lambda j: (0, 0)),
            pl.BlockSpec((ns, D), lambda j: (0, 0)),
            pl.BlockSpec((tv, D), lambda j: (j, 0)),
            pl.BlockSpec((tv, ns), lambda j: (j, 0)),
            pl.BlockSpec((H, D), lambda j: (0, 0)),
            pl.BlockSpec((H, D), lambda j: (0, 0)),
            pl.BlockSpec((1, D), lambda j: (0, 0)),
        ],
        out_specs=pl.BlockSpec((tv, D), lambda j: (j, 0)),
        out_shape=jax.ShapeDtypeStruct((nd, D), jnp.float32),
    )(fs.T, fs, fd, lwt, at, sel, bias.reshape(1, D))


def _sem_body(z0_ref, z1_ref, w1_ref, b1_ref, w2_ref, o_ref):
    w1 = w1_ref[...]
    b1 = b1_ref[...]
    w2 = w2_ref[...]
    z0 = z0_ref[...]
    z1 = z1_ref[...]
    t0 = jnp.tanh(jnp.dot(z0, w1, preferred_element_type=jnp.float32) + b1)
    t1 = jnp.tanh(jnp.dot(z1, w1, preferred_element_type=jnp.float32) + b1)
    n = z0.shape[0]
    s0 = jnp.sum(t0 * w2) / n
    s1 = jnp.sum(t1 * w2) / n
    mx = jnp.maximum(s0, s1)
    e0 = jnp.exp(s0 - mx)
    e1 = jnp.exp(s1 - mx)
    tot = e0 + e1
    o_ref[...] = (e0 / tot) * z0 + (e1 / tot) * z1


def _semantic(z0, z1, p):
    n = z0.shape[0]
    return pl.pallas_call(
        _sem_body,
        out_shape=jax.ShapeDtypeStruct((n, D), jnp.float32),
    )(z0, z1, p['W1'], p['b1'].reshape(1, -1), p['W2'].reshape(1, -1))


def _pred_body(au_ref, tv_ref, w2_ref, b2_ref, o_ref):
    x = jnp.maximum(au_ref[...] + tv_ref[...], 0.0)
    s = jnp.sum(x * w2_ref[...], axis=1, keepdims=True) + b2_ref[...]
    o_ref[...] = jax.nn.sigmoid(s)


def _predict(ha_p, ht_p, u, v, w2, b2):
    e = u.shape[0]
    te = 2048
    ep = ((e + te - 1) // te) * te
    up = jnp.concatenate([u, jnp.zeros((ep - e,), u.dtype)])
    vp = jnp.concatenate([v, jnp.zeros((ep - e,), v.dtype)])
    au = jnp.take(ha_p, up, axis=0)
    tv = jnp.take(ht_p, vp, axis=0)
    out = pl.pallas_call(
        _pred_body,
        grid=(ep // te,),
        in_specs=[
            pl.BlockSpec((te, D), lambda j: (j, 0)),
            pl.BlockSpec((te, D), lambda j: (j, 0)),
            pl.BlockSpec((1, D), lambda j: (0, 0)),
            pl.BlockSpec((1, 1), lambda j: (0, 0)),
        ],
        out_specs=pl.BlockSpec((te, 1), lambda j: (j, 0)),
        out_shape=jax.ShapeDtypeStruct((ep, 1), jnp.float32),
    )(au, tv, w2.reshape(1, D), b2.reshape(1, 1))
    return out[:e, 0]


def _sim_logw(emb):
    n = emb.shape[0]
    d = _pdist(emb)
    flat = d.reshape(-1)
    s = jnp.sort(flat)
    m = n * n - n
    loc = np.float32(10.0 / 100.0) * np.float32(m - 1)
    i0 = int(np.floor(loc))
    g = jnp.float32(loc - np.floor(loc))
    cutoff = s[n + i0] * (1.0 - g) + s[n + i0 + 1] * g
    eye = jnp.eye(n, dtype=bool)
    adj = (d <= cutoff) & (~eye)
    return jnp.where(adj, 0.0, -jnp.inf)


def _count_logw(src, dst, ns, nd):
    cnt = jnp.zeros((nd, ns), jnp.float32).at[dst, src].add(1.0)
    return jnp.log(cnt)


def kernel(protein_embeddings, genomic_embeddings, params,
           amp2target_edges, target2amp_edges, sup_edges, neg_edges):
    p = params
    p_emb = _linear(protein_embeddings, p['W2'], p['b2w'])
    g_emb = _linear(genomic_embeddings, p['W1'], p['b1w'])
    na = p_emb.shape[0]
    nt = g_emb.shape[0]

    lw_p = _sim_logw(p_emb)
    lw_g = _sim_logw(g_emb)
    lw_at = _count_logw(amp2target_edges[0], amp2target_edges[1], na, nt)
    lw_ta = _count_logw(target2amp_edges[0], target2amp_edges[1], nt, na)

    def gat(h_src, h_dst, lwt, gp):
        fs = _linear(h_src, gp['Ws'], gp['bs'])
        fd = _linear(h_dst, gp['Wd'], gp['bd'])
        return _gat_dense(fs, fd, lwt, gp['attn'], gp['bias'])

    z0 = gat(p_emb, p_emb, lw_p, p['gat0'])
    z1 = gat(p_emb, g_emb, lw_at, p['gat1'])
    z2 = gat(g_emb, p_emb, lw_ta, p['gat2'])
    z3 = gat(g_emb, g_emb, lw_g, p['gat3'])

    ha = _semantic(z0, z2, p['p_sem'])
    ht = _semantic(z1, z3, p['g_sem'])

    pr = p['pred']
    a_part = _linear(ha, pr['W1'][:D], pr['b1'])
    t_part = _linear(ht, pr['W1'][D:], jnp.zeros((D,), jnp.float32))
    sup = _predict(a_part, t_part, sup_edges[0], sup_edges[1], pr['W2'], pr['b2'])
    neg = _predict(a_part, t_part, neg_edges[0], neg_edges[1], pr['W2'], pr['b2'])
    return (sup, neg)


# speedup vs baseline: 145.5818x; 2.3894x over previous
import jax
import jax.numpy as jnp
import numpy as np
from jax.experimental import pallas as pl
from jax.experimental.pallas import tpu as pltpu

H = 8
DH = 16
D = H * DH


def _lin_body(x_ref, w_ref, b_ref, o_ref):
    o_ref[...] = jnp.dot(x_ref[...], w_ref[...],
                         preferred_element_type=jnp.float32) + b_ref[...]


def _linear(x, W, b):
    M, K = x.shape
    N = W.shape[1]
    return pl.pallas_call(
        _lin_body,
        out_shape=jax.ShapeDtypeStruct((M, N), jnp.float32),
    )(x, W, b.reshape(1, N))


def _dist_body(xb_ref, emb_ref, sq_ref, o_ref):
    xb = xb_ref[...]
    emb = emb_ref[...]
    g = jax.lax.dot_general(xb, emb, (((1,), (1,)), ((), ())),
                            preferred_element_type=jnp.float32)
    sqb = jnp.sum(xb * xb, axis=1, keepdims=True)
    d2 = sqb + sq_ref[...] - 2.0 * g
    d = jnp.sqrt(jnp.maximum(d2, 0.0))
    tb = xb.shape[0]
    n = emb.shape[0]
    rows = jax.lax.broadcasted_iota(jnp.int32, (tb, n), 0) + pl.program_id(0) * tb
    cols = jax.lax.broadcasted_iota(jnp.int32, (tb, n), 1)
    o_ref[...] = jnp.where(rows == cols, 0.0, d)


def _pdist(emb):
    n, k = emb.shape
    tb = 256
    sq = jnp.sum(emb * emb, axis=1)[None, :]
    return pl.pallas_call(
        _dist_body,
        grid=(n // tb,),
        in_specs=[
            pl.BlockSpec((tb, k), lambda j: (j, 0)),
            pl.BlockSpec((n, k), lambda j: (0, 0)),
            pl.BlockSpec((1, n), lambda j: (0, 0)),
        ],
        out_specs=pl.BlockSpec((tb, n), lambda j: (j, 0)),
        out_shape=jax.ShapeDtypeStruct((n, n), jnp.float32),
    )(emb, emb, sq)


def _gat_body(fsT_ref, fs_ref, fd_ref, lwt_ref, at_ref, sel_ref, b_ref, o_ref):
    fsT = fsT_ref[...]
    fs = fs_ref[...]
    at = at_ref[...]
    sel = sel_ref[...]
    bias = b_ref[...]
    tv = fd_ref.shape[0]

    def step(vi, carry):
        fd_row = fd_ref[pl.ds(vi, 1), :]
        fd_col = fd_row.T
        x = fsT + fd_col
        y = jnp.maximum(x, 0.2 * x)
        logits = jnp.dot(at, y, preferred_element_type=jnp.float32)
        lw = lwt_ref[pl.ds(vi, 1), :]
        logits = logits + lw
        m = jnp.max(logits, axis=1, keepdims=True)
        m = jnp.where(m == -jnp.inf, 0.0, m)
        p = jnp.exp(logits - m)
        den = jnp.sum(p, axis=1, keepdims=True)
        msg = jnp.dot(p, fs, preferred_element_type=jnp.float32)
        num = jnp.sum(msg * sel, axis=0, keepdims=True)
        dsel = jnp.sum(den * sel, axis=0, keepdims=True)
        out_row = jnp.maximum(num / jnp.maximum(dsel, 1e-16) + bias, 0.0)
        o_ref[pl.ds(vi, 1), :] = out_row
        return carry

    jax.lax.fori_loop(0, tv, step, 0)


def _gat_dense(fs, fd, lwt, attn, bias):
    ns = fs.shape[0]
    nd = fd.shape[0]
    tv = 128
    at = attn.reshape(-1)[:, None] * (
        jnp.arange(D)[:, None] // DH == jnp.arange(H)[None, :]).astype(jnp.float32)
    at = at.T
    sel = (jnp.arange(D)[None, :] // DH == jnp.arange(H)[:, None]).astype(jnp.float32)
    return pl.pallas_call(
        _gat_body,
        grid=(nd // tv,),
        in_specs=[
            pl.BlockSpec((D, ns), lambda j: (0, 0)),
            pl.BlockSpec((ns, D), lambda j: (0, 0)),
            pl.BlockSpec((tv, D), lambda j: (j, 0)),
            pl.BlockSpec((tv, ns), lambda j: (j, 0)),
            pl.BlockSpec((H, D), lambda j: (0, 0)),
            pl.BlockSpec((H, D), lambda j: (0, 0)),
            pl.BlockSpec((1, D), lambda j: (0, 0)),
        ],
        out_specs=pl.BlockSpec((tv, D), lambda j: (j, 0)),
        out_shape=jax.ShapeDtypeStruct((nd, D), jnp.float32),
    )(fs.T, fs, fd, lwt, at, sel, bias.reshape(1, D))


def _sem_body(z0_ref, z1_ref, w1_ref, b1_ref, w2_ref, o_ref):
    w1 = w1_ref[...]
    b1 = b1_ref[...]
    w2 = w2_ref[...]
    z0 = z0_ref[...]
    z1 = z1_ref[...]
    t0 = jnp.tanh(jnp.dot(z0, w1, preferred_element_type=jnp.float32) + b1)
    t1 = jnp.tanh(jnp.dot(z1, w1, preferred_element_type=jnp.float32) + b1)
    n = z0.shape[0]
    s0 = jnp.sum(t0 * w2) / n
    s1 = jnp.sum(t1 * w2) / n
    mx = jnp.maximum(s0, s1)
    e0 = jnp.exp(s0 - mx)
    e1 = jnp.exp(s1 - mx)
    tot = e0 + e1
    o_ref[...] = (e0 / tot) * z0 + (e1 / tot) * z1


def _semantic(z0, z1, p):
    n = z0.shape[0]
    return pl.pallas_call(
        _sem_body,
        out_shape=jax.ShapeDtypeStruct((n, D), jnp.float32),
    )(z0, z1, p['W1'], p['b1'].reshape(1, -1), p['W2'].reshape(1, -1))


def _pred_body(au_ref, tv_ref, w2_ref, b2_ref, o_ref):
    x = jnp.maximum(au_ref[...] + tv_ref[...], 0.0)
    s = jnp.sum(x * w2_ref[...], axis=1, keepdims=True) + b2_ref[...]
    o_ref[...] = jax.nn.sigmoid(s)


def _predict(ha_p, ht_p, u, v, w2, b2):
    e = u.shape[0]
    te = 2048
    ep = ((e + te - 1) // te) * te
    up = jnp.concatenate([u, jnp.zeros((ep - e,), u.dtype)])
    vp = jnp.concatenate([v, jnp.zeros((ep - e,), v.dtype)])
    au = jnp.take(ha_p, up, axis=0)
    tv = jnp.take(ht_p, vp, axis=0)
    out = pl.pallas_call(
        _pred_body,
        grid=(ep // te,),
        in_specs=[
            pl.BlockSpec((te, D), lambda j: (j, 0)),
            pl.BlockSpec((te, D), lambda j: (j, 0)),
            pl.BlockSpec((1, D), lambda j: (0, 0)),
            pl.BlockSpec((1, 1), lambda j: (0, 0)),
        ],
        out_specs=pl.BlockSpec((te, 1), lambda j: (j, 0)),
        out_shape=jax.ShapeDtypeStruct((ep, 1), jnp.float32),
    )(au, tv, w2.reshape(1, D), b2.reshape(1, 1))
    return out[:e, 0]


def _radix_count_body(shift):
    def body(bits_ref, qs_ref, o_ref):
        @pl.when(pl.program_id(0) == 0)
        def _init():
            o_ref[...] = jnp.zeros_like(o_ref)

        key = jax.lax.shift_right_logical(bits_ref[...], shift)
        lane = jax.lax.broadcasted_iota(jnp.int32, (1, 128), 1)
        acc = jnp.zeros((1, 128), jnp.float32)
        for s in range(2):
            q0 = qs_ref[s]
            for t in range(4):
                c = jnp.sum((key == q0 + t).astype(jnp.float32))
                acc = acc + jnp.where(lane == 8 * s + t, c, 0.0)
        o_ref[...] = o_ref[...] + acc
    return body


def _order_stats(d, k1):
    n2 = d.size
    bits = jax.lax.bitcast_convert_type(d, jnp.int32)
    tb = 256
    nrows, ncols = d.shape
    base = jnp.zeros((2,), jnp.int32)
    r = jnp.array([k1, k1 + 1], jnp.float32)
    for shift in range(30, -2, -2):
        counts = pl.pallas_call(
            _radix_count_body(shift),
            grid=(nrows // tb,),
            in_specs=[
                pl.BlockSpec((tb, ncols), lambda j: (j, 0)),
                pl.BlockSpec(memory_space=pltpu.SMEM),
            ],
            out_specs=pl.BlockSpec((1, 128), lambda j: (0, 0)),
            out_shape=jax.ShapeDtypeStruct((1, 128), jnp.float32),
        )(bits, base)
        nb = []
        nr = []
        for s in range(2):
            c = counts[0, 8 * s:8 * s + 4]
            cz = jnp.concatenate([jnp.zeros((1,), jnp.float32), jnp.cumsum(c)])
            j = jnp.sum((cz[1:] <= r[s]).astype(jnp.int32))
            nr.append(r[s] - cz[j])
            b = base[s] + j
            nb.append(b if shift == 0 else b << 2)
        base = jnp.stack(nb)
        r = jnp.stack(nr)
    v = jax.lax.bitcast_convert_type(base, jnp.float32)
    return v[0], v[1]


def _sim_logw(emb):
    n = emb.shape[0]
    d = _pdist(emb)
    m = n * n - n
    loc = np.float32(10.0 / 100.0) * np.float32(m - 1)
    i0 = int(np.floor(loc))
    g = jnp.float32(loc - np.floor(loc))
    v1, v2 = _order_stats(d, n + i0)
    cutoff = v1 * (1.0 - g) + v2 * g
    eye = jnp.eye(n, dtype=bool)
    adj = (d <= cutoff) & (~eye)
    return jnp.where(adj, 0.0, -jnp.inf)


def _count_logw(src, dst, ns, nd):
    cnt = jnp.zeros((nd, ns), jnp.float32).at[dst, src].add(1.0)
    return jnp.log(cnt)


def kernel(protein_embeddings, genomic_embeddings, params,
           amp2target_edges, target2amp_edges, sup_edges, neg_edges):
    p = params
    p_emb = _linear(protein_embeddings, p['W2'], p['b2w'])
    g_emb = _linear(genomic_embeddings, p['W1'], p['b1w'])
    na = p_emb.shape[0]
    nt = g_emb.shape[0]

    lw_p = _sim_logw(p_emb)
    lw_g = _sim_logw(g_emb)
    lw_at = _count_logw(amp2target_edges[0], amp2target_edges[1], na, nt)
    lw_ta = _count_logw(target2amp_edges[0], target2amp_edges[1], nt, na)

    def gat(h_src, h_dst, lwt, gp):
        fs = _linear(h_src, gp['Ws'], gp['bs'])
        fd = _linear(h_dst, gp['Wd'], gp['bd'])
        return _gat_dense(fs, fd, lwt, gp['attn'], gp['bias'])

    z0 = gat(p_emb, p_emb, lw_p, p['gat0'])
    z1 = gat(p_emb, g_emb, lw_at, p['gat1'])
    z2 = gat(g_emb, p_emb, lw_ta, p['gat2'])
    z3 = gat(g_emb, g_emb, lw_g, p['gat3'])

    ha = _semantic(z0, z2, p['p_sem'])
    ht = _semantic(z1, z3, p['g_sem'])

    pr = p['pred']
    a_part = _linear(ha, pr['W1'][:D], pr['b1'])
    t_part = _linear(ht, pr['W1'][D:], jnp.zeros((D,), jnp.float32))
    sup = _predict(a_part, t_part, sup_edges[0], sup_edges[1], pr['W2'], pr['b2'])
    neg = _predict(a_part, t_part, neg_edges[0], neg_edges[1], pr['W2'], pr['b2'])
    return (sup, neg)
